# baseline (speedup 1.0000x reference)
import functools

import jax
import jax.numpy as jnp
from jax import lax
from jax.experimental import pallas as pl
from jax.experimental.pallas import tpu as pltpu
from jax.experimental.pallas import tpu_sc as plsc

_RADIUS = 1.6
_K = 2048
_N = 65536
_M = 128
_R = 512
_C = 128
_KROWS = _K // _C
_NB = _N // _K


def _lex_lt(ka, ia, kb, ib):
    return (ka < kb) | ((ka == kb) & (ia < ib))


def _stage(K, I, up, j, rid, cid):
    if j < _C:
        lower = (cid & j) == 0
        Ku = pltpu.roll(K, _C - j, 1)
        Kd = pltpu.roll(K, j, 1)
        Iu = pltpu.roll(I, _C - j, 1)
        Id = pltpu.roll(I, j, 1)
    else:
        s = j // _C
        lower = (rid & s) == 0
        Ku = jnp.concatenate([K[s:], K[:s]], 0)
        Kd = jnp.concatenate([K[-s:], K[:-s]], 0)
        Iu = jnp.concatenate([I[s:], I[:s]], 0)
        Id = jnp.concatenate([I[-s:], I[:-s]], 0)
    Kp = jnp.where(lower, Ku, Kd)
    Ip = jnp.where(lower, Iu, Id)
    want_self_min = up == lower
    self_lt = _lex_lt(K, I, Kp, Ip)
    take = jnp.logical_xor(self_lt, want_self_min)
    return jnp.where(take, Kp, K), jnp.where(take, Ip, I)


_BROWS = 512
_NSTEPS = _R // _BROWS


def _dist_sort_body(pts_ref, rois_ref, mindis_ref, idx_ref, ksc, isc):
    pid = pl.program_id(0)

    if True:
        px = pts_ref[0]
        py = pts_ref[1]
        pz = pts_ref[2]

        def one_roi(j):
            cx = rois_ref[0, j]
            cy = rois_ref[1, j]
            cz = rois_ref[2, j]
            hx = rois_ref[3, j] * 0.5
            hy = rois_ref[4, j] * 0.5
            hz = rois_ref[5, j] * 0.5
            rj = jnp.sqrt((hx * hx + hy * hy) + hz * hz)
            dx = px - cx
            dy = py - cy
            dz = pz - cz
            dist = jnp.sqrt((dx * dx + dy * dy) + dz * dz)
            return dist, rj

        def roi_step(q, carry):
            m, rn = carry
            d0, r0 = one_roi(4 * q)
            d1, r1 = one_roi(4 * q + 1)
            d2, r2 = one_roi(4 * q + 2)
            d3, r3 = one_roi(4 * q + 3)
            lt1 = d1 < d0
            d01 = jnp.where(lt1, d1, d0)
            r01 = jnp.where(lt1, r1, r0)
            lt2 = d3 < d2
            d23 = jnp.where(lt2, d3, d2)
            r23 = jnp.where(lt2, r3, r2)
            lt3 = d23 < d01
            dn = jnp.where(lt3, d23, d01)
            rn4 = jnp.where(lt3, r23, r01)
            upd = dn < m
            return jnp.where(upd, dn, m), jnp.where(upd, rn4, rn)

        m0 = jnp.full((_BROWS, _C), jnp.inf, jnp.float32)
        m, rn = lax.fori_loop(0, _M // 4, roi_step, (m0, m0))
        mindis_ref[...] = m

        mask = m < rn + _RADIUS
        K = jnp.where(mask, lax.bitcast_convert_type(m, jnp.uint32),
                      jnp.uint32(0xFF000000))
        rid = lax.broadcasted_iota(jnp.int32, (_BROWS, _C), 0)
        cid = lax.broadcasted_iota(jnp.int32, (_BROWS, _C), 1)
        I = pid * (_BROWS * _C) + rid * _C + cid

        kk = 2
        while kk <= _K:
            if kk < _C:
                up = (cid & kk) == 0
            else:
                up = (rid & (kk // _C)) == 0
            j = kk // 2
            while j >= 1:
                K, I = _stage(K, I, up, j, rid, cid)
                j //= 2
            kk *= 2

        ksc[pl.ds(pid * _BROWS, _BROWS), :] = K
        isc[pl.ds(pid * _BROWS, _BROWS), :] = I

    @pl.when(pid == _NSTEPS - 1)
    def _phase_merge():
        K = ksc[...]
        I = isc[...]
        nb = _NB
        while nb > 1:
            rows = nb * _KROWS // 2
            Kr = K.reshape(nb // 2, 2 * _KROWS, _C)
            Ir = I.reshape(nb // 2, 2 * _KROWS, _C)
            a_lt = _lex_lt(Kr[:, :_KROWS, :], Ir[:, :_KROWS, :],
                           Kr[:, _KROWS:, :], Ir[:, _KROWS:, :])
            K = jnp.where(a_lt, Kr[:, :_KROWS, :],
                          Kr[:, _KROWS:, :]).reshape(rows, _C)
            I = jnp.where(a_lt, Ir[:, :_KROWS, :],
                          Ir[:, _KROWS:, :]).reshape(rows, _C)
            nb //= 2
            rid = lax.broadcasted_iota(jnp.int32, (rows, _C), 0)
            cid = lax.broadcasted_iota(jnp.int32, (rows, _C), 1)
            up = ((rid >> 4) & 1) == 0
            j = _K // 2
            while j >= 1:
                K, I = _stage(K, I, up, j, rid, cid)
                j //= 2
        idx_ref[...] = I


@jax.jit
def _dist_sort(pts_t, rois):
    return pl.pallas_call(
        _dist_sort_body,
        grid=(_NSTEPS,),
        in_specs=[
            pl.BlockSpec((3, _BROWS, _C), lambda i: (0, i, 0)),
            pl.BlockSpec(memory_space=pltpu.SMEM),
        ],
        out_specs=[
            pl.BlockSpec((_BROWS, _C), lambda i: (i, 0)),
            pl.BlockSpec((_KROWS, _C), lambda i: (0, 0)),
        ],
        out_shape=[
            jax.ShapeDtypeStruct((_R, _C), jnp.float32),
            jax.ShapeDtypeStruct((_KROWS, _C), jnp.int32),
        ],
        scratch_shapes=[
            pltpu.VMEM((_R, _C), jnp.uint32),
            pltpu.VMEM((_R, _C), jnp.int32),
        ],
    )(pts_t, rois)


_NW = 32
_BPW = _K // _NW


def _sc_gather_body(px_hbm, py_hbm, pz_hbm, r_hbm, idx_hbm,
                    x_out, y_out, z_out, r_out,
                    idx_v, xv, yv, zv, rv, sem):
    wid = lax.axis_index("s") * 2 + lax.axis_index("c")
    base = wid * _BPW
    pltpu.sync_copy(idx_hbm.at[pl.ds(base, _BPW)], idx_v)
    cps = [
        pltpu.async_copy(px_hbm.at[idx_v], xv, sem),
        pltpu.async_copy(py_hbm.at[idx_v], yv, sem),
        pltpu.async_copy(pz_hbm.at[idx_v], zv, sem),
        pltpu.async_copy(r_hbm.at[idx_v], rv, sem),
    ]
    for cp in cps:
        cp.wait()
    pltpu.sync_copy(xv, x_out.at[pl.ds(base, _BPW)])
    pltpu.sync_copy(yv, y_out.at[pl.ds(base, _BPW)])
    pltpu.sync_copy(zv, z_out.at[pl.ds(base, _BPW)])
    pltpu.sync_copy(rv, r_out.at[pl.ds(base, _BPW)])


@jax.jit
def _sc_gather(px, py, pz, r_flat, idx):
    vec = jax.ShapeDtypeStruct((_K,), jnp.float32)
    f = functools.partial(
        pl.kernel,
        out_type=(vec, vec, vec, vec),
        mesh=plsc.VectorSubcoreMesh(core_axis_name="c", subcore_axis_name="s"),
        scratch_types=[
            pltpu.VMEM((_BPW,), jnp.int32),
            pltpu.VMEM((_BPW,), jnp.float32),
            pltpu.VMEM((_BPW,), jnp.float32),
            pltpu.VMEM((_BPW,), jnp.float32),
            pltpu.VMEM((_BPW,), jnp.float32),
            pltpu.SemaphoreType.DMA,
        ],
    )(_sc_gather_body)
    return f(px, py, pz, r_flat, idx)


def kernel(points, rois):
    pts_t = points.T.reshape(3, _R, _C)
    mindis, topidx = _dist_sort(pts_t, rois.T)
    idx = topidx.reshape(_K)
    r_flat = mindis.reshape(_N)
    flat = pts_t.reshape(3, _N)
    x, y, z, r = _sc_gather(flat[0], flat[1], flat[2], r_flat, idx)
    return jnp.stack([x, y, z, r], axis=1)


# speedup vs baseline: 1.0174x; 1.0174x over previous
import functools

import jax
import jax.numpy as jnp
from jax import lax
from jax.experimental import pallas as pl
from jax.experimental.pallas import tpu as pltpu
from jax.experimental.pallas import tpu_sc as plsc

_RADIUS = 1.6
_K = 2048
_N = 65536
_M = 128
_R = 512
_C = 128
_KROWS = _K // _C
_NB = _N // _K


def _lex_lt(ka, ia, kb, ib):
    return (ka < kb) | ((ka == kb) & (ia < ib))


def _stage(K, I, up, j, rid, cid):
    if j < _C:
        lower = (cid & j) == 0
        Ku = pltpu.roll(K, _C - j, 1)
        Kd = pltpu.roll(K, j, 1)
        Iu = pltpu.roll(I, _C - j, 1)
        Id = pltpu.roll(I, j, 1)
    else:
        s = j // _C
        lower = (rid & s) == 0
        Ku = jnp.concatenate([K[s:], K[:s]], 0)
        Kd = jnp.concatenate([K[-s:], K[:-s]], 0)
        Iu = jnp.concatenate([I[s:], I[:s]], 0)
        Id = jnp.concatenate([I[-s:], I[:-s]], 0)
    Kp = jnp.where(lower, Ku, Kd)
    Ip = jnp.where(lower, Iu, Id)
    want_self_min = up == lower
    self_lt = _lex_lt(K, I, Kp, Ip)
    take = jnp.logical_xor(self_lt, want_self_min)
    return jnp.where(take, Kp, K), jnp.where(take, Ip, I)


_BROWS = 256
_NSTEPS = _R // _BROWS


def _dist_sort_body(pts_ref, rois_ref, mindis_ref, idx_ref, ksc, isc):
    pid = pl.program_id(0)

    if True:
        px = pts_ref[0]
        py = pts_ref[1]
        pz = pts_ref[2]

        def one_roi(j):
            cx = rois_ref[0, j]
            cy = rois_ref[1, j]
            cz = rois_ref[2, j]
            hx = rois_ref[3, j] * 0.5
            hy = rois_ref[4, j] * 0.5
            hz = rois_ref[5, j] * 0.5
            rj = jnp.sqrt((hx * hx + hy * hy) + hz * hz)
            dx = px - cx
            dy = py - cy
            dz = pz - cz
            dist = jnp.sqrt((dx * dx + dy * dy) + dz * dz)
            return dist, rj

        def roi_step(q, carry):
            m, rn = carry
            d0, r0 = one_roi(4 * q)
            d1, r1 = one_roi(4 * q + 1)
            d2, r2 = one_roi(4 * q + 2)
            d3, r3 = one_roi(4 * q + 3)
            lt1 = d1 < d0
            d01 = jnp.where(lt1, d1, d0)
            r01 = jnp.where(lt1, r1, r0)
            lt2 = d3 < d2
            d23 = jnp.where(lt2, d3, d2)
            r23 = jnp.where(lt2, r3, r2)
            lt3 = d23 < d01
            dn = jnp.where(lt3, d23, d01)
            rn4 = jnp.where(lt3, r23, r01)
            upd = dn < m
            return jnp.where(upd, dn, m), jnp.where(upd, rn4, rn)

        m0 = jnp.full((_BROWS, _C), jnp.inf, jnp.float32)
        m, rn = lax.fori_loop(0, _M // 4, roi_step, (m0, m0))
        mindis_ref[...] = m

        mask = m < rn + _RADIUS
        K = jnp.where(mask, lax.bitcast_convert_type(m, jnp.uint32),
                      jnp.uint32(0xFF000000))
        rid = lax.broadcasted_iota(jnp.int32, (_BROWS, _C), 0)
        cid = lax.broadcasted_iota(jnp.int32, (_BROWS, _C), 1)
        I = pid * (_BROWS * _C) + rid * _C + cid

        kk = 2
        while kk <= _K:
            if kk < _C:
                up = (cid & kk) == 0
            else:
                up = (rid & (kk // _C)) == 0
            j = kk // 2
            while j >= 1:
                K, I = _stage(K, I, up, j, rid, cid)
                j //= 2
            kk *= 2

        ksc[pl.ds(pid * _BROWS, _BROWS), :] = K
        isc[pl.ds(pid * _BROWS, _BROWS), :] = I

    @pl.when(pid == _NSTEPS - 1)
    def _phase_merge():
        K = ksc[...]
        I = isc[...]
        nb = _NB
        while nb > 1:
            rows = nb * _KROWS // 2
            Kr = K.reshape(nb // 2, 2 * _KROWS, _C)
            Ir = I.reshape(nb // 2, 2 * _KROWS, _C)
            a_lt = _lex_lt(Kr[:, :_KROWS, :], Ir[:, :_KROWS, :],
                           Kr[:, _KROWS:, :], Ir[:, _KROWS:, :])
            K = jnp.where(a_lt, Kr[:, :_KROWS, :],
                          Kr[:, _KROWS:, :]).reshape(rows, _C)
            I = jnp.where(a_lt, Ir[:, :_KROWS, :],
                          Ir[:, _KROWS:, :]).reshape(rows, _C)
            nb //= 2
            rid = lax.broadcasted_iota(jnp.int32, (rows, _C), 0)
            cid = lax.broadcasted_iota(jnp.int32, (rows, _C), 1)
            up = ((rid >> 4) & 1) == 0
            j = _K // 2
            while j >= 1:
                K, I = _stage(K, I, up, j, rid, cid)
                j //= 2
        idx_ref[...] = I


@jax.jit
def _dist_sort(pts_t, rois):
    return pl.pallas_call(
        _dist_sort_body,
        grid=(_NSTEPS,),
        in_specs=[
            pl.BlockSpec((3, _BROWS, _C), lambda i: (0, i, 0)),
            pl.BlockSpec(memory_space=pltpu.SMEM),
        ],
        out_specs=[
            pl.BlockSpec((_BROWS, _C), lambda i: (i, 0)),
            pl.BlockSpec((_KROWS, _C), lambda i: (0, 0)),
        ],
        out_shape=[
            jax.ShapeDtypeStruct((_R, _C), jnp.float32),
            jax.ShapeDtypeStruct((_KROWS, _C), jnp.int32),
        ],
        scratch_shapes=[
            pltpu.VMEM((_R, _C), jnp.uint32),
            pltpu.VMEM((_R, _C), jnp.int32),
        ],
    )(pts_t, rois)


_NW = 32
_BPW = _K // _NW


def _sc_gather_body(px_hbm, py_hbm, pz_hbm, r_hbm, idx_hbm,
                    x_out, y_out, z_out, r_out,
                    idx_v, xv, yv, zv, rv, sem):
    wid = lax.axis_index("s") * 2 + lax.axis_index("c")
    base = wid * _BPW
    pltpu.sync_copy(idx_hbm.at[pl.ds(base, _BPW)], idx_v)
    cps = [
        pltpu.async_copy(px_hbm.at[idx_v], xv, sem),
        pltpu.async_copy(py_hbm.at[idx_v], yv, sem),
        pltpu.async_copy(pz_hbm.at[idx_v], zv, sem),
        pltpu.async_copy(r_hbm.at[idx_v], rv, sem),
    ]
    for cp in cps:
        cp.wait()
    pltpu.sync_copy(xv, x_out.at[pl.ds(base, _BPW)])
    pltpu.sync_copy(yv, y_out.at[pl.ds(base, _BPW)])
    pltpu.sync_copy(zv, z_out.at[pl.ds(base, _BPW)])
    pltpu.sync_copy(rv, r_out.at[pl.ds(base, _BPW)])


@jax.jit
def _sc_gather(px, py, pz, r_flat, idx):
    vec = jax.ShapeDtypeStruct((_K,), jnp.float32)
    f = functools.partial(
        pl.kernel,
        out_type=(vec, vec, vec, vec),
        mesh=plsc.VectorSubcoreMesh(core_axis_name="c", subcore_axis_name="s"),
        scratch_types=[
            pltpu.VMEM((_BPW,), jnp.int32),
            pltpu.VMEM((_BPW,), jnp.float32),
            pltpu.VMEM((_BPW,), jnp.float32),
            pltpu.VMEM((_BPW,), jnp.float32),
            pltpu.VMEM((_BPW,), jnp.float32),
            pltpu.SemaphoreType.DMA,
        ],
    )(_sc_gather_body)
    return f(px, py, pz, r_flat, idx)


def kernel(points, rois):
    pts_t = points.T.reshape(3, _R, _C)
    mindis, topidx = _dist_sort(pts_t, rois.T)
    idx = topidx.reshape(_K)
    r_flat = mindis.reshape(_N)
    flat = pts_t.reshape(3, _N)
    x, y, z, r = _sc_gather(flat[0], flat[1], flat[2], r_flat, idx)
    return jnp.stack([x, y, z, r], axis=1)
